# baseline (speedup 1.0000x reference)
import functools

import jax
import jax.numpy as jnp
from jax import lax
from jax.experimental import pallas as pl
from jax.experimental.pallas import tpu as pltpu
from jax.experimental.pallas import tpu_sc as plsc

D = 64
DP = 128


@functools.lru_cache(maxsize=None)
def _build_transpose(V: int):
    info = plsc.get_sparse_core_info()
    NC, NS = info.num_cores, info.num_subcores
    NW = NC * NS
    NB_FULL = V // DP - 1
    V_TAIL = NB_FULL * DP
    N_I = (NB_FULL + NW - 1) // NW

    mesh = plsc.VectorSubcoreMesh(core_axis_name="c", subcore_axis_name="s")

    @functools.partial(
        pl.kernel,
        mesh=mesh,
        compiler_params=pltpu.CompilerParams(
            use_tc_tiling_on_sc=True, needs_layout_passes=False
        ),
        out_type=jax.ShapeDtypeStruct((V, DP), jnp.float32),
        scratch_types=[
            pltpu.VMEM((D, 192), jnp.float32),
            pltpu.VMEM((192, DP), jnp.float32),
            pltpu.SemaphoreType.DMA,
        ],
    )
    def transpose_kernel(tt_hbm, out_hbm, src_v, dst_v, rsem):
        wid = lax.axis_index("s") * NC + lax.axis_index("c")
        IOTA = lax.iota(jnp.int32, 16)

        def do_block(base, width):
            waits = []
            for dh in range(8):
                waits.append(pltpu.async_copy(
                    tt_hbm.at[pl.ds(dh * 8, 8), pl.ds(base, width)],
                    src_v.at[pl.ds(dh * 8, 8), pl.ds(0, width)],
                    rsem,
                ))
            for h in waits:
                h.wait()
            for d in range(D):
                col = jnp.full((16,), d, jnp.int32)
                for g in range(width // 16):
                    x = src_v[d, pl.ds(g * 16, 16)]
                    plsc.store_scatter(dst_v, [IOTA + g * 16, col], x)
            pltpu.sync_copy(
                dst_v.at[pl.ds(0, width), :], out_hbm.at[pl.ds(base, width)]
            )

        def body(i, carry):
            b = wid + i * NW

            @pl.when(b < NB_FULL)
            def _():
                do_block(b * DP, DP)

            return carry

        lax.fori_loop(0, N_I, body, jnp.int32(0))

        @pl.when(wid == NB_FULL % NW)
        def _():
            do_block(V_TAIL, V - V_TAIL)

    return transpose_kernel


@functools.lru_cache(maxsize=None)
def _build_gather(T: int, V: int):
    info = plsc.get_sparse_core_info()
    NC, NS = info.num_cores, info.num_subcores
    NW = NC * NS
    assert T % NW == 0
    b_per_w = T // NW
    C = 400
    assert b_per_w % C == 0
    n_chunks = b_per_w // C

    mesh = plsc.VectorSubcoreMesh(core_axis_name="c", subcore_axis_name="s")

    @functools.partial(
        pl.kernel,
        mesh=mesh,
        compiler_params=pltpu.CompilerParams(use_tc_tiling_on_sc=True),
        out_type=jax.ShapeDtypeStruct((T, DP), jnp.float32),
        scratch_types=[
            pltpu.VMEM((b_per_w,), jnp.int32),
            pltpu.VMEM((2, C, DP), jnp.float32),
            pltpu.SemaphoreType.DMA,
            pltpu.SemaphoreType.DMA,
        ],
    )
    def gather_kernel(table_hbm, idx_hbm, out_hbm, idx_v, rows_v, gsem, ssem):
        wid = lax.axis_index("s") * NC + lax.axis_index("c")
        base = wid * b_per_w
        pltpu.sync_copy(idx_hbm.at[pl.ds(base, b_per_w)], idx_v)
        gathers = [None] * n_chunks
        scatters = [None] * n_chunks
        gathers[0] = pltpu.async_copy(
            table_hbm.at[idx_v.at[pl.ds(0, C)]], rows_v.at[0], gsem
        )
        for j in range(n_chunks):
            if j + 1 < n_chunks:
                if j >= 1:
                    scatters[j - 1].wait()
                gathers[j + 1] = pltpu.async_copy(
                    table_hbm.at[idx_v.at[pl.ds((j + 1) * C, C)]],
                    rows_v.at[(j + 1) % 2],
                    gsem,
                )
            gathers[j].wait()
            scatters[j] = pltpu.async_copy(
                rows_v.at[j % 2], out_hbm.at[pl.ds(base + j * C, C)], ssem
            )
        scatters[n_chunks - 2].wait()
        scatters[n_chunks - 1].wait()

    return gather_kernel


def kernel(caption, table):
    B, L = caption.shape
    T = B * L
    V = table.shape[0]
    idx = jnp.swapaxes(caption, 0, 1).reshape(T).astype(jnp.int32)
    tt = jnp.swapaxes(table, 0, 1)
    table_rm = _build_transpose(V)(tt)
    out = _build_gather(T, V)(table_rm, idx)
    out64 = out[:, :D]
    return jnp.swapaxes(out64.reshape(L, B, D), 0, 1)


# speedup vs baseline: 1.6795x; 1.6795x over previous
import functools

import jax
import jax.numpy as jnp
from jax import lax
from jax.experimental import pallas as pl
from jax.experimental.pallas import tpu as pltpu
from jax.experimental.pallas import tpu_sc as plsc

D = 64
DP = 128


@functools.lru_cache(maxsize=None)
def _build_transpose(V: int):
    info = plsc.get_sparse_core_info()
    NC, NS = info.num_cores, info.num_subcores
    NW = NC * NS
    NB_FULL = V // DP - 1
    V_TAIL = NB_FULL * DP
    N_I = (NB_FULL + NW - 1) // NW

    mesh = plsc.VectorSubcoreMesh(core_axis_name="c", subcore_axis_name="s")

    @functools.partial(
        pl.kernel,
        mesh=mesh,
        compiler_params=pltpu.CompilerParams(
            use_tc_tiling_on_sc=True, needs_layout_passes=False
        ),
        out_type=jax.ShapeDtypeStruct((V, DP), jnp.float32),
        scratch_types=[
            pltpu.VMEM((2, D, DP), jnp.float32),
            pltpu.VMEM((2, DP, DP), jnp.float32),
            pltpu.VMEM((D, 192), jnp.float32),
            pltpu.VMEM((192, DP), jnp.float32),
            pltpu.SemaphoreType.DMA,
            pltpu.SemaphoreType.DMA,
            pltpu.SemaphoreType.DMA,
            pltpu.SemaphoreType.DMA,
            pltpu.SemaphoreType.DMA,
        ],
    )
    def transpose_kernel(tt_hbm, out_hbm, src_v, dst_v, tsrc_v, tdst_v,
                         rsem0, rsem1, wsem0, wsem1, tsem):
        wid = lax.axis_index("s") * NC + lax.axis_index("c")
        IOTA = lax.iota(jnp.int32, 16)
        rsems = (rsem0, rsem1)
        wsems = (wsem0, wsem1)

        def issue_reads(base, par):
            for dh in range(8):
                pltpu.async_copy(
                    tt_hbm.at[pl.ds(dh * 8, 8), pl.ds(base, DP)],
                    src_v.at[par, pl.ds(dh * 8, 8), :],
                    rsems[par],
                )

        def transpose(sv, dv, width):
            for g in range(width // 16):
                rows = IOTA + g * 16
                for d0 in range(0, D, 8):
                    xs = [sv[d0 + k, pl.ds(g * 16, 16)] for k in range(8)]
                    for k in range(8):
                        col = jnp.full((16,), d0 + k, jnp.int32)
                        plsc.store_scatter(dv, [rows, col], xs[k])

        def body(i2, carry):
            for par in range(2):
                j = 2 * i2 + par
                b = wid + j * NW

                @pl.when(b < NB_FULL)
                def _():
                    bn = b + NW

                    @pl.when(bn < NB_FULL)
                    def _():
                        issue_reads(bn * DP, (par + 1) % 2)

                    pltpu.make_async_copy(
                        tt_hbm.at[pl.ds(0, D), pl.ds(0, DP)],
                        src_v.at[par], rsems[par],
                    ).wait()

                    @pl.when(j >= 2)
                    def _():
                        pltpu.make_async_copy(
                            out_hbm.at[pl.ds(0, DP)],
                            dst_v.at[par], wsems[par],
                        ).wait()

                    transpose(src_v.at[par], dst_v.at[par], DP)
                    pltpu.async_copy(
                        dst_v.at[par], out_hbm.at[pl.ds(b * DP, DP)],
                        wsems[par],
                    )

            return carry

        issue_reads(wid * DP, 0)
        lax.fori_loop(0, (N_I + 1) // 2, body, jnp.int32(0))

        n_b = (NB_FULL - wid + NW - 1) // NW

        @pl.when(n_b >= 1)
        def _():
            pltpu.make_async_copy(
                out_hbm.at[pl.ds(0, DP)], dst_v.at[0], wsem0
            ).wait()

        @pl.when(n_b >= 2)
        def _():
            pltpu.make_async_copy(
                out_hbm.at[pl.ds(0, DP)], dst_v.at[1], wsem1
            ).wait()

        @pl.when(wid == NB_FULL % NW)
        def _():
            waits = []
            for dh in range(8):
                waits.append(pltpu.async_copy(
                    tt_hbm.at[pl.ds(dh * 8, 8), pl.ds(V_TAIL, V - V_TAIL)],
                    tsrc_v.at[pl.ds(dh * 8, 8), :],
                    tsem,
                ))
            for h in waits:
                h.wait()
            transpose(tsrc_v, tdst_v, V - V_TAIL)
            pltpu.sync_copy(tdst_v, out_hbm.at[pl.ds(V_TAIL, V - V_TAIL)])

    return transpose_kernel


@functools.lru_cache(maxsize=None)
def _build_gather(T: int, V: int):
    info = plsc.get_sparse_core_info()
    NC, NS = info.num_cores, info.num_subcores
    NW = NC * NS
    assert T % NW == 0
    b_per_w = T // NW
    C = 400
    assert b_per_w % C == 0
    n_chunks = b_per_w // C

    mesh = plsc.VectorSubcoreMesh(core_axis_name="c", subcore_axis_name="s")

    @functools.partial(
        pl.kernel,
        mesh=mesh,
        compiler_params=pltpu.CompilerParams(use_tc_tiling_on_sc=True),
        out_type=jax.ShapeDtypeStruct((T, DP), jnp.float32),
        scratch_types=[
            pltpu.VMEM((b_per_w,), jnp.int32),
            pltpu.VMEM((2, C, DP), jnp.float32),
            pltpu.SemaphoreType.DMA,
            pltpu.SemaphoreType.DMA,
        ],
    )
    def gather_kernel(table_hbm, idx_hbm, out_hbm, idx_v, rows_v, gsem, ssem):
        wid = lax.axis_index("s") * NC + lax.axis_index("c")
        base = wid * b_per_w
        pltpu.sync_copy(idx_hbm.at[pl.ds(base, b_per_w)], idx_v)
        gathers = [None] * n_chunks
        scatters = [None] * n_chunks
        gathers[0] = pltpu.async_copy(
            table_hbm.at[idx_v.at[pl.ds(0, C)]], rows_v.at[0], gsem
        )
        for j in range(n_chunks):
            if j + 1 < n_chunks:
                if j >= 1:
                    scatters[j - 1].wait()
                gathers[j + 1] = pltpu.async_copy(
                    table_hbm.at[idx_v.at[pl.ds((j + 1) * C, C)]],
                    rows_v.at[(j + 1) % 2],
                    gsem,
                )
            gathers[j].wait()
            scatters[j] = pltpu.async_copy(
                rows_v.at[j % 2], out_hbm.at[pl.ds(base + j * C, C)], ssem
            )
        scatters[n_chunks - 2].wait()
        scatters[n_chunks - 1].wait()

    return gather_kernel


def kernel(caption, table):
    B, L = caption.shape
    T = B * L
    V = table.shape[0]
    idx = jnp.swapaxes(caption, 0, 1).reshape(T).astype(jnp.int32)
    tt = jnp.swapaxes(table, 0, 1)
    table_rm = _build_transpose(V)(tt)
    out = _build_gather(T, V)(table_rm, idx)
    out64 = out[:, :D]
    return jnp.swapaxes(out64.reshape(L, B, D), 0, 1)
